# baseline (speedup 1.0000x reference)
import functools

import jax
import jax.numpy as jnp
from jax import lax
from jax.experimental import pallas as pl
from jax.experimental.pallas import tpu as pltpu
from jax.experimental.pallas import tpu_sc as plsc

B = 8; NP = 512; EP = 1024; D = 128; H = 4
N = B * NP; E = B * EP
KV = NP // 2; KE = EP // 2
DH = D // H


def _nt(a, b):
    return jax.lax.dot_general(a, b, (((1,), (1,)), ((), ())))


def _tn(a, b):
    return jax.lax.dot_general(a, b, (((0,), (0,)), ((), ())))


def _gcn_deg_mm_body(np_, x_ref, agg_ref, dst_ref, w_ref, o_ref):
    g = pl.program_id(0)
    ne = dst_ref.shape[2]
    dst_row = dst_ref[0]
    node_col = jax.lax.broadcasted_iota(jnp.int32, (np_, ne), 0) + g * np_
    onehot = (jnp.broadcast_to(dst_row, (np_, ne)) == node_col).astype(jnp.float32)
    deg = jnp.sum(onehot, axis=1, keepdims=True)
    xb = (agg_ref[0] + x_ref[0]) / (deg + 1.0)
    o_ref[0] = jax.nn.relu(xb @ w_ref[...])


def _gcn_mm(x, agg, dst, w, np_, ne):
    nb = x.shape[0] // np_
    x3 = x.reshape(nb, np_, D)
    agg3 = agg.reshape(nb, np_, D)
    dst3 = dst.reshape(nb, 1, ne)
    out = pl.pallas_call(
        functools.partial(_gcn_deg_mm_body, np_),
        grid=(nb,),
        in_specs=[pl.BlockSpec((1, np_, D), lambda i: (i, 0, 0)),
                  pl.BlockSpec((1, np_, D), lambda i: (i, 0, 0)),
                  pl.BlockSpec((1, 1, ne), lambda i: (i, 0, 0)),
                  pl.BlockSpec((D, D), lambda i: (0, 0))],
        out_specs=pl.BlockSpec((1, np_, D), lambda i: (i, 0, 0)),
        out_shape=jax.ShapeDtypeStruct((nb, np_, D), jnp.float32),
    )(x3, agg3, dst3, w)
    return out.reshape(x.shape[0], D)


def _mix_body(h_ref, t_ref, w_ref, o_ref):
    o_ref[...] = jax.nn.relu(h_ref[...] + t_ref[...] @ w_ref[...])


def _mix_mm(h, t, w, rows_per_block=1024):
    n = h.shape[0]
    rb = rows_per_block
    return pl.pallas_call(
        _mix_body,
        grid=(n // rb,),
        in_specs=[pl.BlockSpec((rb, D), lambda i: (i, 0)),
                  pl.BlockSpec((rb, D), lambda i: (i, 0)),
                  pl.BlockSpec((D, D), lambda i: (0, 0))],
        out_specs=pl.BlockSpec((rb, D), lambda i: (i, 0)),
        out_shape=jax.ShapeDtypeStruct((n, D), jnp.float32),
    )(h, t, w)


def _qkv_body(x_ref, wq_ref, wk_ref, wv_ref, q_ref, k_ref, v_ref):
    x = x_ref[0]
    q_ref[0] = x @ wq_ref[...]
    k_ref[0] = x @ wk_ref[...]
    v_ref[0] = x @ wv_ref[...]


def _qkv_flat_body(x_ref, wq_ref, wk_ref, wv_ref, q_ref, k_ref, v_ref):
    x = x_ref[...]
    q_ref[...] = x @ wq_ref[...]
    k_ref[...] = x @ wk_ref[...]
    v_ref[...] = x @ wv_ref[...]


def _qkv(xd, Wq, Wk, Wv, rb=2048):
    Bq, L, _ = xd.shape
    n = Bq * L
    x2 = xd.reshape(n, D)
    w_spec = pl.BlockSpec((D, D), lambda i: (0, 0))
    x_spec = pl.BlockSpec((rb, D), lambda i: (i, 0))
    q, k, v = pl.pallas_call(
        _qkv_flat_body,
        grid=(n // rb,),
        in_specs=[x_spec, w_spec, w_spec, w_spec],
        out_specs=[x_spec, x_spec, x_spec],
        out_shape=[jax.ShapeDtypeStruct((n, D), jnp.float32)] * 3,
    )(x2, Wq, Wk, Wv)
    return q.reshape(Bq, L, D), k.reshape(Bq, L, D), v.reshape(Bq, L, D)


_NW = 32
_GRW = E // _NW
_GADD = None


def _gadd_body(vh_hbm, si_hbm, di_hbm, out_hbm, ia, ib, bufa, bufb, sema, semb):
    c = lax.axis_index("c")
    s = lax.axis_index("s")
    w = s * 2 + c
    pltpu.sync_copy(si_hbm.at[w], ia)
    pltpu.sync_copy(di_hbm.at[w], ib)
    cps = []
    for j in range(_GRW // 128):
        cps.append(pltpu.async_copy(vh_hbm.at[ia.at[j]], bufa.at[pl.ds(j * 128, 128)], sema))
    for cp in cps:
        cp.wait()
    cps = []
    for j in range(_GRW // 128):
        cps.append(pltpu.async_copy(vh_hbm.at[ib.at[j]], bufa.at[pl.ds(j * 128, 128)], semb, add=True))
    for cp in cps:
        cp.wait()
    pltpu.sync_copy(bufa, out_hbm.at[pl.ds(w * _GRW, _GRW)])


def _gather_add(vh, src3, dst3):
    global _GADD
    if _GADD is None:
        _GADD = pl.kernel(
            _gadd_body,
            out_type=jax.ShapeDtypeStruct((E, D), jnp.float32),
            mesh=plsc.VectorSubcoreMesh(core_axis_name="c", subcore_axis_name="s"),
            scratch_types=[
                pltpu.VMEM((_GRW // 128, 128), jnp.int32),
                pltpu.VMEM((_GRW // 128, 128), jnp.int32),
                pltpu.VMEM((_GRW, D), jnp.float32),
                pltpu.VMEM((_GRW, D), jnp.float32),
                pltpu.SemaphoreType.DMA,
                pltpu.SemaphoreType.DMA,
            ],
        )
    return _GADD(vh, src3, dst3)


def _att_core(q, k, v):
    Bq, L, _ = q.shape
    qh = q.reshape(Bq, L, H, DH).transpose(0, 2, 1, 3)
    kh = k.reshape(Bq, L, H, DH).transpose(0, 2, 1, 3)
    vh = v.reshape(Bq, L, H, DH).transpose(0, 2, 1, 3)
    scores = jnp.einsum('bhqd,bhkd->bhqk', qh, kh) / jnp.sqrt(jnp.float32(DH))
    att = jax.nn.softmax(scores, axis=-1)
    out = jnp.einsum('bhqk,bhkd->bhqd', att, vh).transpose(0, 2, 1, 3).reshape(Bq, L, D)
    return out


def _wo_body(x_ref, wo_ref, o_ref):
    o_ref[0] = x_ref[0] @ wo_ref[...]


def _wo_flat_body(x_ref, wo_ref, o_ref):
    o_ref[...] = x_ref[...] @ wo_ref[...]


def _wo_mm(xd, Wo, rb=2048):
    Bq, L, _ = xd.shape
    n = Bq * L
    out = pl.pallas_call(
        _wo_flat_body,
        grid=(n // rb,),
        in_specs=[pl.BlockSpec((rb, D), lambda i: (i, 0)),
                  pl.BlockSpec((D, D), lambda i: (0, 0))],
        out_specs=pl.BlockSpec((rb, D), lambda i: (i, 0)),
        out_shape=jax.ShapeDtypeStruct((n, D), jnp.float32),
    )(xd.reshape(n, D), Wo)
    return out.reshape(Bq, L, D)


def _pool_body(K, x_ref, s_ref, sub_ref, perm_ref):
    L = x_ref.shape[1]
    x = x_ref[0]
    s_col = s_ref[0]
    s_row = jnp.transpose(s_col)
    sc = jnp.broadcast_to(s_col, (L, L))
    sr = jnp.broadcast_to(s_row, (L, L))
    ii = jax.lax.broadcasted_iota(jnp.int32, (L, L), 1)
    jj = jax.lax.broadcasted_iota(jnp.int32, (L, L), 0)
    beats = (sr > sc) | ((sr == sc) & (ii < jj))
    rank_col = jnp.sum(beats.astype(jnp.float32), axis=1, keepdims=True)
    r_lk = jax.lax.broadcasted_iota(jnp.int32, (L, K), 1).astype(jnp.float32)
    oh = (jnp.broadcast_to(rank_col, (L, K)) == r_lk).astype(jnp.float32)
    sub = _tn(oh, x)
    vals = _tn(oh, s_col)
    iota_col = jax.lax.broadcasted_iota(jnp.int32, (L, 1), 0).astype(jnp.float32)
    perm = _tn(oh, iota_col)
    sub_ref[0] = sub * jnp.tanh(vals)
    perm_ref[0] = perm.astype(jnp.int32)


def _pool(xd, scores, K):
    Bq, L, _ = xd.shape
    return pl.pallas_call(
        functools.partial(_pool_body, K),
        grid=(Bq,),
        in_specs=[pl.BlockSpec((1, L, D), lambda g: (g, 0, 0)),
                  pl.BlockSpec((1, L, 1), lambda g: (g, 0, 0))],
        out_specs=[pl.BlockSpec((1, K, D), lambda g: (g, 0, 0)),
                   pl.BlockSpec((1, K, 1), lambda g: (g, 0, 0))],
        out_shape=[jax.ShapeDtypeStruct((Bq, K, D), jnp.float32),
                   jax.ShapeDtypeStruct((Bq, K, 1), jnp.int32)],
    )(xd, scores)


def _encoder(xd, Wq, Wk, Wv, Wo):
    q, k, v = _qkv(xd, Wq, Wk, Wv)
    mid = _att_core(q, k, v)
    return _wo_mm(mid, Wo)


def _score(enc, pv):
    pn = pv / (jnp.linalg.norm(pv) + 1e-12)
    return (enc @ pn)[..., None]


def kernel(sparse_x, edge_index, batch, e_x, e_edge_index, e_batch,
           Wv_g, We_g, Wev, Wve, Wq, Wk, Wva, Wo, p):
    src, dst = edge_index[0], edge_index[1]
    esrc, edst = e_edge_index[0], e_edge_index[1]

    v_agg = jax.ops.segment_sum(sparse_x[src], dst, num_segments=N)
    e_agg = jax.ops.segment_sum(e_x[esrc], edst, num_segments=E)
    v_h = _gcn_mm(sparse_x, v_agg, dst, Wv_g, NP, EP)
    e_h = _gcn_mm(e_x, e_agg, edst, We_g, EP, 2 * EP)

    t_v = jax.ops.segment_sum(e_h, src, num_segments=N) + jax.ops.segment_sum(e_h, dst, num_segments=N)
    g2 = _gather_add(v_h, src.reshape(_NW, _GRW // 128, 128), dst.reshape(_NW, _GRW // 128, 128))

    v_out = _mix_mm(v_h, t_v, Wev)
    e_out = _mix_mm(e_h, g2, Wve)
    v_xd = v_out.reshape(B, NP, D)
    e_xd = e_out.reshape(B, EP, D)

    enc1 = _encoder(v_xd, Wq[0], Wk[0], Wva[0], Wo[0])
    v1, p1 = _pool(enc1, _score(enc1, p[0]), KV)
    enc2 = _encoder(enc1, Wq[1], Wk[1], Wva[1], Wo[1])
    v2, p2 = _pool(enc2, _score(enc2, p[1]), KV)
    ee1 = _encoder(e_xd, Wq[2], Wk[2], Wva[2], Wo[2])
    e1, q1 = _pool(ee1, _score(ee1, p[2]), KE)
    ee2 = _encoder(ee1, Wq[3], Wk[3], Wva[3], Wo[3])
    e2, q2 = _pool(ee2, _score(ee2, p[3]), KE)

    out = jnp.concatenate([v1.reshape(-1, D), v2.reshape(-1, D),
                           e1.reshape(-1, D), e2.reshape(-1, D)], axis=0)
    return (out, p1.reshape(B, KV), p2.reshape(B, KV),
            q1.reshape(B, KE), q2.reshape(B, KE))


# speedup vs baseline: 1.0438x; 1.0438x over previous
import functools

import jax
import jax.numpy as jnp
from jax import lax
from jax.experimental import pallas as pl
from jax.experimental.pallas import tpu as pltpu
from jax.experimental.pallas import tpu_sc as plsc

B = 8; NP = 512; EP = 1024; D = 128; H = 4
N = B * NP; E = B * EP
KV = NP // 2; KE = EP // 2
DH = D // H


def _nt(a, b):
    return jax.lax.dot_general(a, b, (((1,), (1,)), ((), ())))


def _tn(a, b):
    return jax.lax.dot_general(a, b, (((0,), (0,)), ((), ())))


def _gcn_deg_mm_body(np_, x_ref, agg_ref, dst_ref, w_ref, o_ref):
    g = pl.program_id(0)
    ne = dst_ref.shape[2]
    dst_row = dst_ref[0]
    node_col = jax.lax.broadcasted_iota(jnp.int32, (np_, ne), 0) + g * np_
    onehot = (jnp.broadcast_to(dst_row, (np_, ne)) == node_col).astype(jnp.float32)
    deg = jnp.sum(onehot, axis=1, keepdims=True)
    xb = (agg_ref[0] + x_ref[0]) / (deg + 1.0)
    o_ref[0] = jax.nn.relu(xb @ w_ref[...])


def _gcn_mm(x, agg, dst, w, np_, ne):
    nb = x.shape[0] // np_
    x3 = x.reshape(nb, np_, D)
    agg3 = agg.reshape(nb, np_, D)
    dst3 = dst.reshape(nb, 1, ne)
    out = pl.pallas_call(
        functools.partial(_gcn_deg_mm_body, np_),
        grid=(nb,),
        in_specs=[pl.BlockSpec((1, np_, D), lambda i: (i, 0, 0)),
                  pl.BlockSpec((1, np_, D), lambda i: (i, 0, 0)),
                  pl.BlockSpec((1, 1, ne), lambda i: (i, 0, 0)),
                  pl.BlockSpec((D, D), lambda i: (0, 0))],
        out_specs=pl.BlockSpec((1, np_, D), lambda i: (i, 0, 0)),
        out_shape=jax.ShapeDtypeStruct((nb, np_, D), jnp.float32),
    )(x3, agg3, dst3, w)
    return out.reshape(x.shape[0], D)


def _mix_body(h_ref, t_ref, w_ref, o_ref):
    o_ref[...] = jax.nn.relu(h_ref[...] + t_ref[...] @ w_ref[...])


def _mix_mm(h, t, w, rows_per_block=1024):
    n = h.shape[0]
    rb = rows_per_block
    return pl.pallas_call(
        _mix_body,
        grid=(n // rb,),
        in_specs=[pl.BlockSpec((rb, D), lambda i: (i, 0)),
                  pl.BlockSpec((rb, D), lambda i: (i, 0)),
                  pl.BlockSpec((D, D), lambda i: (0, 0))],
        out_specs=pl.BlockSpec((rb, D), lambda i: (i, 0)),
        out_shape=jax.ShapeDtypeStruct((n, D), jnp.float32),
    )(h, t, w)


def _qkv_body(x_ref, wq_ref, wk_ref, wv_ref, q_ref, k_ref, v_ref):
    x = x_ref[0]
    q_ref[0] = x @ wq_ref[...]
    k_ref[0] = x @ wk_ref[...]
    v_ref[0] = x @ wv_ref[...]


def _qkv(xd, Wq, Wk, Wv):
    Bq, L, _ = xd.shape
    w_spec = pl.BlockSpec((D, D), lambda g: (0, 0))
    x_spec = pl.BlockSpec((1, L, D), lambda g: (g, 0, 0))
    return pl.pallas_call(
        _qkv_body,
        grid=(Bq,),
        in_specs=[x_spec, w_spec, w_spec, w_spec],
        out_specs=[x_spec, x_spec, x_spec],
        out_shape=[jax.ShapeDtypeStruct((Bq, L, D), jnp.float32)] * 3,
    )(xd, Wq, Wk, Wv)


_NW = 32
_GRW = E // _NW
_GADD = None


def _gadd_body(vh_hbm, si_hbm, di_hbm, out_hbm, ia, ib, bufa, bufb, sema, semb):
    c = lax.axis_index("c")
    s = lax.axis_index("s")
    w = s * 2 + c
    pltpu.sync_copy(si_hbm.at[w], ia)
    pltpu.sync_copy(di_hbm.at[w], ib)
    cps = []
    for j in range(_GRW // 128):
        cps.append(pltpu.async_copy(vh_hbm.at[ia.at[j]], bufa.at[pl.ds(j * 128, 128)], sema))
    for cp in cps:
        cp.wait()
    cps = []
    for j in range(_GRW // 128):
        cps.append(pltpu.async_copy(vh_hbm.at[ib.at[j]], bufa.at[pl.ds(j * 128, 128)], semb, add=True))
    for cp in cps:
        cp.wait()
    pltpu.sync_copy(bufa, out_hbm.at[pl.ds(w * _GRW, _GRW)])


def _gather_add(vh, src3, dst3):
    global _GADD
    if _GADD is None:
        _GADD = pl.kernel(
            _gadd_body,
            out_type=jax.ShapeDtypeStruct((E, D), jnp.float32),
            mesh=plsc.VectorSubcoreMesh(core_axis_name="c", subcore_axis_name="s"),
            scratch_types=[
                pltpu.VMEM((_GRW // 128, 128), jnp.int32),
                pltpu.VMEM((_GRW // 128, 128), jnp.int32),
                pltpu.VMEM((_GRW, D), jnp.float32),
                pltpu.VMEM((_GRW, D), jnp.float32),
                pltpu.SemaphoreType.DMA,
                pltpu.SemaphoreType.DMA,
            ],
        )
    return _GADD(vh, src3, dst3)


def _att_core(q, k, v):
    Bq, L, _ = q.shape
    qh = q.reshape(Bq, L, H, DH).transpose(0, 2, 1, 3)
    kh = k.reshape(Bq, L, H, DH).transpose(0, 2, 1, 3)
    vh = v.reshape(Bq, L, H, DH).transpose(0, 2, 1, 3)
    scores = jnp.einsum('bhqd,bhkd->bhqk', qh, kh) / jnp.sqrt(jnp.float32(DH))
    att = jax.nn.softmax(scores, axis=-1)
    out = jnp.einsum('bhqk,bhkd->bhqd', att, vh).transpose(0, 2, 1, 3).reshape(Bq, L, D)
    return out


def _wo_body(x_ref, wo_ref, o_ref):
    o_ref[0] = x_ref[0] @ wo_ref[...]


def _wo_mm(xd, Wo):
    Bq, L, _ = xd.shape
    x_spec = pl.BlockSpec((1, L, D), lambda g: (g, 0, 0))
    return pl.pallas_call(
        _wo_body,
        grid=(Bq,),
        in_specs=[x_spec, pl.BlockSpec((D, D), lambda g: (0, 0))],
        out_specs=x_spec,
        out_shape=jax.ShapeDtypeStruct((Bq, L, D), jnp.float32),
    )(xd, Wo)


def _pool_body(K, x_ref, s_ref, sub_ref, perm_ref):
    L = x_ref.shape[1]
    x = x_ref[0]
    s_col = s_ref[0]
    s_row = jnp.transpose(s_col)
    sc = jnp.broadcast_to(s_col, (L, L))
    sr = jnp.broadcast_to(s_row, (L, L))
    ii = jax.lax.broadcasted_iota(jnp.int32, (L, L), 1)
    jj = jax.lax.broadcasted_iota(jnp.int32, (L, L), 0)
    beats = (sr > sc) | ((sr == sc) & (ii < jj))
    rank_col = jnp.sum(beats.astype(jnp.float32), axis=1, keepdims=True)
    r_lk = jax.lax.broadcasted_iota(jnp.int32, (L, K), 1).astype(jnp.float32)
    oh = (jnp.broadcast_to(rank_col, (L, K)) == r_lk).astype(jnp.float32)
    sub = _tn(oh, x)
    vals = _tn(oh, s_col)
    iota_col = jax.lax.broadcasted_iota(jnp.int32, (L, 1), 0).astype(jnp.float32)
    perm = _tn(oh, iota_col)
    sub_ref[0] = sub * jnp.tanh(vals)
    perm_ref[0] = perm.astype(jnp.int32)


def _pool(xd, scores, K):
    Bq, L, _ = xd.shape
    return pl.pallas_call(
        functools.partial(_pool_body, K),
        grid=(Bq,),
        in_specs=[pl.BlockSpec((1, L, D), lambda g: (g, 0, 0)),
                  pl.BlockSpec((1, L, 1), lambda g: (g, 0, 0))],
        out_specs=[pl.BlockSpec((1, K, D), lambda g: (g, 0, 0)),
                   pl.BlockSpec((1, K, 1), lambda g: (g, 0, 0))],
        out_shape=[jax.ShapeDtypeStruct((Bq, K, D), jnp.float32),
                   jax.ShapeDtypeStruct((Bq, K, 1), jnp.int32)],
    )(xd, scores)


def _encoder(xd, Wq, Wk, Wv, Wo):
    q, k, v = _qkv(xd, Wq, Wk, Wv)
    mid = _att_core(q, k, v)
    return _wo_mm(mid, Wo)


def _score(enc, pv):
    pn = pv / (jnp.linalg.norm(pv) + 1e-12)
    return (enc @ pn)[..., None]


def kernel(sparse_x, edge_index, batch, e_x, e_edge_index, e_batch,
           Wv_g, We_g, Wev, Wve, Wq, Wk, Wva, Wo, p):
    src, dst = edge_index[0], edge_index[1]
    esrc, edst = e_edge_index[0], e_edge_index[1]

    v_agg = jax.ops.segment_sum(sparse_x[src], dst, num_segments=N)
    e_agg = jax.ops.segment_sum(e_x[esrc], edst, num_segments=E)
    v_h = _gcn_mm(sparse_x, v_agg, dst, Wv_g, NP, EP)
    e_h = _gcn_mm(e_x, e_agg, edst, We_g, EP, 2 * EP)

    t_v = jax.ops.segment_sum(e_h, src, num_segments=N) + jax.ops.segment_sum(e_h, dst, num_segments=N)
    g2 = _gather_add(v_h, src.reshape(_NW, _GRW // 128, 128), dst.reshape(_NW, _GRW // 128, 128))

    v_out = _mix_mm(v_h, t_v, Wev)
    e_out = _mix_mm(e_h, g2, Wve)
    v_xd = v_out.reshape(B, NP, D)
    e_xd = e_out.reshape(B, EP, D)

    enc1 = _encoder(v_xd, Wq[0], Wk[0], Wva[0], Wo[0])
    v1, p1 = _pool(enc1, _score(enc1, p[0]), KV)
    enc2 = _encoder(enc1, Wq[1], Wk[1], Wva[1], Wo[1])
    v2, p2 = _pool(enc2, _score(enc2, p[1]), KV)
    ee1 = _encoder(e_xd, Wq[2], Wk[2], Wva[2], Wo[2])
    e1, q1 = _pool(ee1, _score(ee1, p[2]), KE)
    ee2 = _encoder(ee1, Wq[3], Wk[3], Wva[3], Wo[3])
    e2, q2 = _pool(ee2, _score(ee2, p[3]), KE)

    out = jnp.concatenate([v1.reshape(-1, D), v2.reshape(-1, D),
                           e1.reshape(-1, D), e2.reshape(-1, D)], axis=0)
    return (out, p1.reshape(B, KV), p2.reshape(B, KV),
            q1.reshape(B, KE), q2.reshape(B, KE))


# speedup vs baseline: 1.0529x; 1.0087x over previous
import functools

import jax
import jax.numpy as jnp
from jax import lax
from jax.experimental import pallas as pl
from jax.experimental.pallas import tpu as pltpu
from jax.experimental.pallas import tpu_sc as plsc

B = 8; NP = 512; EP = 1024; D = 128; H = 4
N = B * NP; E = B * EP
KV = NP // 2; KE = EP // 2
DH = D // H


def _nt(a, b):
    return jax.lax.dot_general(a, b, (((1,), (1,)), ((), ())))


def _tn(a, b):
    return jax.lax.dot_general(a, b, (((0,), (0,)), ((), ())))


def _gcn_deg_mm_body(np_, x_ref, agg_ref, dst_ref, w_ref, o_ref):
    g = pl.program_id(0)
    ne = dst_ref.shape[2]
    dst_row = dst_ref[0]
    node_col = jax.lax.broadcasted_iota(jnp.int32, (np_, ne), 0) + g * np_
    onehot = (jnp.broadcast_to(dst_row, (np_, ne)) == node_col).astype(jnp.float32)
    deg = jnp.sum(onehot, axis=1, keepdims=True)
    xb = (agg_ref[0] + x_ref[0]) / (deg + 1.0)
    o_ref[0] = jax.nn.relu(xb @ w_ref[...])


def _gcn_mm(x, agg, dst, w, np_, ne):
    nb = x.shape[0] // np_
    x3 = x.reshape(nb, np_, D)
    agg3 = agg.reshape(nb, np_, D)
    dst3 = dst.reshape(nb, 1, ne)
    out = pl.pallas_call(
        functools.partial(_gcn_deg_mm_body, np_),
        grid=(nb,),
        in_specs=[pl.BlockSpec((1, np_, D), lambda i: (i, 0, 0)),
                  pl.BlockSpec((1, np_, D), lambda i: (i, 0, 0)),
                  pl.BlockSpec((1, 1, ne), lambda i: (i, 0, 0)),
                  pl.BlockSpec((D, D), lambda i: (0, 0))],
        out_specs=pl.BlockSpec((1, np_, D), lambda i: (i, 0, 0)),
        out_shape=jax.ShapeDtypeStruct((nb, np_, D), jnp.float32),
    )(x3, agg3, dst3, w)
    return out.reshape(x.shape[0], D)


def _mix_body(h_ref, t_ref, w_ref, o_ref):
    o_ref[...] = jax.nn.relu(h_ref[...] + t_ref[...] @ w_ref[...])


def _mix_mm(h, t, w, rows_per_block=1024):
    n = h.shape[0]
    rb = rows_per_block
    return pl.pallas_call(
        _mix_body,
        grid=(n // rb,),
        in_specs=[pl.BlockSpec((rb, D), lambda i: (i, 0)),
                  pl.BlockSpec((rb, D), lambda i: (i, 0)),
                  pl.BlockSpec((D, D), lambda i: (0, 0))],
        out_specs=pl.BlockSpec((rb, D), lambda i: (i, 0)),
        out_shape=jax.ShapeDtypeStruct((n, D), jnp.float32),
    )(h, t, w)


def _qkv_body(x_ref, wq_ref, wk_ref, wv_ref, q_ref, k_ref, v_ref):
    x = x_ref[0]
    q_ref[0] = x @ wq_ref[...]
    k_ref[0] = x @ wk_ref[...]
    v_ref[0] = x @ wv_ref[...]


def _qkv(xd, Wq, Wk, Wv):
    Bq, L, _ = xd.shape
    w_spec = pl.BlockSpec((D, D), lambda g: (0, 0))
    x_spec = pl.BlockSpec((1, L, D), lambda g: (g, 0, 0))
    return pl.pallas_call(
        _qkv_body,
        grid=(Bq,),
        in_specs=[x_spec, w_spec, w_spec, w_spec],
        out_specs=[x_spec, x_spec, x_spec],
        out_shape=[jax.ShapeDtypeStruct((Bq, L, D), jnp.float32)] * 3,
    )(xd, Wq, Wk, Wv)


def _mixqkv_body(h_ref, t_ref, wm_ref, wq_ref, wk_ref, wv_ref,
                 q_ref, k_ref, v_ref):
    x = jax.nn.relu(h_ref[0] + t_ref[0] @ wm_ref[...])
    q_ref[0] = x @ wq_ref[...]
    k_ref[0] = x @ wk_ref[...]
    v_ref[0] = x @ wv_ref[...]


def _mix_qkv(h, t3, Wm, Wq, Wk, Wv, L):
    Bq = h.shape[0] // L
    h3 = h.reshape(Bq, L, D)
    if t3.ndim == 2:
        t3 = t3.reshape(Bq, L, D)
    w_spec = pl.BlockSpec((D, D), lambda g: (0, 0))
    x_spec = pl.BlockSpec((1, L, D), lambda g: (g, 0, 0))
    return pl.pallas_call(
        _mixqkv_body,
        grid=(Bq,),
        in_specs=[x_spec, x_spec, w_spec, w_spec, w_spec, w_spec],
        out_specs=[x_spec, x_spec, x_spec],
        out_shape=[jax.ShapeDtypeStruct((Bq, L, D), jnp.float32)] * 3,
    )(h3, t3, Wm, Wq, Wk, Wv)


_NW = 32
_GRW = E // _NW
_GADD = None


def _gadd_body(vh_hbm, si_hbm, di_hbm, out_hbm, ia, ib, bufa, bufb, sema, semb):
    c = lax.axis_index("c")
    s = lax.axis_index("s")
    w = s * 2 + c
    pltpu.sync_copy(si_hbm.at[w], ia)
    pltpu.sync_copy(di_hbm.at[w], ib)
    cps = []
    for j in range(_GRW // 128):
        cps.append(pltpu.async_copy(vh_hbm.at[ia.at[j]], bufa.at[pl.ds(j * 128, 128)], sema))
    for cp in cps:
        cp.wait()
    cps = []
    for j in range(_GRW // 128):
        cps.append(pltpu.async_copy(vh_hbm.at[ib.at[j]], bufa.at[pl.ds(j * 128, 128)], semb, add=True))
    for cp in cps:
        cp.wait()
    pltpu.sync_copy(bufa, out_hbm.at[pl.ds(w * _GRW, _GRW)])


def _gather_add(vh, src3, dst3):
    global _GADD
    if _GADD is None:
        _GADD = pl.kernel(
            _gadd_body,
            out_type=jax.ShapeDtypeStruct((E, D), jnp.float32),
            mesh=plsc.VectorSubcoreMesh(core_axis_name="c", subcore_axis_name="s"),
            scratch_types=[
                pltpu.VMEM((_GRW // 128, 128), jnp.int32),
                pltpu.VMEM((_GRW // 128, 128), jnp.int32),
                pltpu.VMEM((_GRW, D), jnp.float32),
                pltpu.VMEM((_GRW, D), jnp.float32),
                pltpu.SemaphoreType.DMA,
                pltpu.SemaphoreType.DMA,
            ],
        )
    return _GADD(vh, src3, dst3)


def _att_core(q, k, v):
    Bq, L, _ = q.shape
    qh = q.reshape(Bq, L, H, DH).transpose(0, 2, 1, 3)
    kh = k.reshape(Bq, L, H, DH).transpose(0, 2, 1, 3)
    vh = v.reshape(Bq, L, H, DH).transpose(0, 2, 1, 3)
    scores = jnp.einsum('bhqd,bhkd->bhqk', qh, kh) / jnp.sqrt(jnp.float32(DH))
    att = jax.nn.softmax(scores, axis=-1)
    out = jnp.einsum('bhqk,bhkd->bhqd', att, vh).transpose(0, 2, 1, 3).reshape(Bq, L, D)
    return out


def _wo_body(x_ref, wo_ref, o_ref):
    o_ref[0] = x_ref[0] @ wo_ref[...]


def _wo_mm(xd, Wo):
    Bq, L, _ = xd.shape
    x_spec = pl.BlockSpec((1, L, D), lambda g: (g, 0, 0))
    return pl.pallas_call(
        _wo_body,
        grid=(Bq,),
        in_specs=[x_spec, pl.BlockSpec((D, D), lambda g: (0, 0))],
        out_specs=x_spec,
        out_shape=jax.ShapeDtypeStruct((Bq, L, D), jnp.float32),
    )(xd, Wo)


def _pool_body(K, x_ref, s_ref, sub_ref, perm_ref):
    L = x_ref.shape[1]
    x = x_ref[0]
    s_col = s_ref[0]
    s_row = jnp.transpose(s_col)
    sc = jnp.broadcast_to(s_col, (L, L))
    sr = jnp.broadcast_to(s_row, (L, L))
    ii = jax.lax.broadcasted_iota(jnp.int32, (L, L), 1)
    jj = jax.lax.broadcasted_iota(jnp.int32, (L, L), 0)
    beats = (sr > sc) | ((sr == sc) & (ii < jj))
    rank_col = jnp.sum(beats.astype(jnp.float32), axis=1, keepdims=True)
    r_lk = jax.lax.broadcasted_iota(jnp.int32, (L, K), 1).astype(jnp.float32)
    oh = (jnp.broadcast_to(rank_col, (L, K)) == r_lk).astype(jnp.float32)
    sub = _tn(oh, x)
    vals = _tn(oh, s_col)
    iota_col = jax.lax.broadcasted_iota(jnp.int32, (L, 1), 0).astype(jnp.float32)
    perm = _tn(oh, iota_col)
    sub_ref[0] = sub * jnp.tanh(vals)
    perm_ref[0] = perm.astype(jnp.int32)


def _pool(xd, scores, K):
    Bq, L, _ = xd.shape
    return pl.pallas_call(
        functools.partial(_pool_body, K),
        grid=(Bq,),
        in_specs=[pl.BlockSpec((1, L, D), lambda g: (g, 0, 0)),
                  pl.BlockSpec((1, L, 1), lambda g: (g, 0, 0))],
        out_specs=[pl.BlockSpec((1, K, D), lambda g: (g, 0, 0)),
                   pl.BlockSpec((1, K, 1), lambda g: (g, 0, 0))],
        out_shape=[jax.ShapeDtypeStruct((Bq, K, D), jnp.float32),
                   jax.ShapeDtypeStruct((Bq, K, 1), jnp.int32)],
    )(xd, scores)


def _encoder(xd, Wq, Wk, Wv, Wo):
    q, k, v = _qkv(xd, Wq, Wk, Wv)
    mid = _att_core(q, k, v)
    return _wo_mm(mid, Wo)


def _score(enc, pv):
    pn = pv / (jnp.linalg.norm(pv) + 1e-12)
    return (enc @ pn)[..., None]


def kernel(sparse_x, edge_index, batch, e_x, e_edge_index, e_batch,
           Wv_g, We_g, Wev, Wve, Wq, Wk, Wva, Wo, p):
    src, dst = edge_index[0], edge_index[1]
    esrc, edst = e_edge_index[0], e_edge_index[1]

    v_agg = jax.ops.segment_sum(sparse_x[src], dst, num_segments=N)
    e_agg = jax.ops.segment_sum(e_x[esrc], edst, num_segments=E)
    v_h = _gcn_mm(sparse_x, v_agg, dst, Wv_g, NP, EP)
    e_h = _gcn_mm(e_x, e_agg, edst, We_g, EP, 2 * EP)

    t_v = jax.ops.segment_sum(e_h, src, num_segments=N) + jax.ops.segment_sum(e_h, dst, num_segments=N)
    g2 = _gather_add(v_h, src.reshape(_NW, _GRW // 128, 128), dst.reshape(_NW, _GRW // 128, 128))

    qv, kv, vv = _mix_qkv(v_h, t_v, Wev, Wq[0], Wk[0], Wva[0], NP)
    enc1 = _wo_mm(_att_core(qv, kv, vv), Wo[0])
    v1, p1 = _pool(enc1, _score(enc1, p[0]), KV)
    enc2 = _encoder(enc1, Wq[1], Wk[1], Wva[1], Wo[1])
    v2, p2 = _pool(enc2, _score(enc2, p[1]), KV)
    qe, ke, ve = _mix_qkv(e_h, g2, Wve, Wq[2], Wk[2], Wva[2], EP)
    ee1 = _wo_mm(_att_core(qe, ke, ve), Wo[2])
    e1, q1 = _pool(ee1, _score(ee1, p[2]), KE)
    ee2 = _encoder(ee1, Wq[3], Wk[3], Wva[3], Wo[3])
    e2, q2 = _pool(ee2, _score(ee2, p[3]), KE)

    out = jnp.concatenate([v1.reshape(-1, D), v2.reshape(-1, D),
                           e1.reshape(-1, D), e2.reshape(-1, D)], axis=0)
    return (out, p1.reshape(B, KV), p2.reshape(B, KV),
            q1.reshape(B, KE), q2.reshape(B, KE))


# speedup vs baseline: 1.0672x; 1.0137x over previous
import functools

import jax
import jax.numpy as jnp
from jax import lax
from jax.experimental import pallas as pl
from jax.experimental.pallas import tpu as pltpu
from jax.experimental.pallas import tpu_sc as plsc

B = 8; NP = 512; EP = 1024; D = 128; H = 4
N = B * NP; E = B * EP
KV = NP // 2; KE = EP // 2
DH = D // H


def _nt(a, b):
    return jax.lax.dot_general(a, b, (((1,), (1,)), ((), ())))


def _tn(a, b):
    return jax.lax.dot_general(a, b, (((0,), (0,)), ((), ())))


def _gcn_deg_mm_body(np_, x_ref, agg_ref, dst_ref, w_ref, o_ref):
    g = pl.program_id(0)
    ne = dst_ref.shape[2]
    dst_row = dst_ref[0]
    node_col = jax.lax.broadcasted_iota(jnp.int32, (np_, ne), 0) + g * np_
    onehot = (jnp.broadcast_to(dst_row, (np_, ne)) == node_col).astype(jnp.float32)
    deg = jnp.sum(onehot, axis=1, keepdims=True)
    xb = (agg_ref[0] + x_ref[0]) / (deg + 1.0)
    o_ref[0] = jax.nn.relu(xb @ w_ref[...])


def _gcn_mm(x, agg, dst, w, np_, ne):
    nb = x.shape[0] // np_
    x3 = x.reshape(nb, np_, D)
    agg3 = agg.reshape(nb, np_, D)
    dst3 = dst.reshape(nb, 1, ne)
    out = pl.pallas_call(
        functools.partial(_gcn_deg_mm_body, np_),
        grid=(nb,),
        in_specs=[pl.BlockSpec((1, np_, D), lambda i: (i, 0, 0)),
                  pl.BlockSpec((1, np_, D), lambda i: (i, 0, 0)),
                  pl.BlockSpec((1, 1, ne), lambda i: (i, 0, 0)),
                  pl.BlockSpec((D, D), lambda i: (0, 0))],
        out_specs=pl.BlockSpec((1, np_, D), lambda i: (i, 0, 0)),
        out_shape=jax.ShapeDtypeStruct((nb, np_, D), jnp.float32),
    )(x3, agg3, dst3, w)
    return out.reshape(x.shape[0], D)


def _mix_body(h_ref, t_ref, w_ref, o_ref):
    o_ref[...] = jax.nn.relu(h_ref[...] + t_ref[...] @ w_ref[...])


def _mix_mm(h, t, w, rows_per_block=1024):
    n = h.shape[0]
    rb = rows_per_block
    return pl.pallas_call(
        _mix_body,
        grid=(n // rb,),
        in_specs=[pl.BlockSpec((rb, D), lambda i: (i, 0)),
                  pl.BlockSpec((rb, D), lambda i: (i, 0)),
                  pl.BlockSpec((D, D), lambda i: (0, 0))],
        out_specs=pl.BlockSpec((rb, D), lambda i: (i, 0)),
        out_shape=jax.ShapeDtypeStruct((n, D), jnp.float32),
    )(h, t, w)


def _qkv_body(x_ref, wq_ref, wk_ref, wv_ref, q_ref, k_ref, v_ref):
    x = x_ref[0]
    q_ref[0] = x @ wq_ref[...]
    k_ref[0] = x @ wk_ref[...]
    v_ref[0] = x @ wv_ref[...]


def _qkv(xd, Wq, Wk, Wv):
    Bq, L, _ = xd.shape
    w_spec = pl.BlockSpec((D, D), lambda g: (0, 0))
    x_spec = pl.BlockSpec((1, L, D), lambda g: (g, 0, 0))
    return pl.pallas_call(
        _qkv_body,
        grid=(Bq,),
        in_specs=[x_spec, w_spec, w_spec, w_spec],
        out_specs=[x_spec, x_spec, x_spec],
        out_shape=[jax.ShapeDtypeStruct((Bq, L, D), jnp.float32)] * 3,
    )(xd, Wq, Wk, Wv)


def _mixqkv_body(h_ref, t_ref, wm_ref, wq_ref, wk_ref, wv_ref,
                 q_ref, k_ref, v_ref):
    x = jax.nn.relu(h_ref[0] + t_ref[0] @ wm_ref[...])
    q_ref[0] = x @ wq_ref[...]
    k_ref[0] = x @ wk_ref[...]
    v_ref[0] = x @ wv_ref[...]


def _mix_qkv(h, t3, Wm, Wq, Wk, Wv, L):
    Bq = h.shape[0] // L
    h3 = h.reshape(Bq, L, D)
    if t3.ndim == 2:
        t3 = t3.reshape(Bq, L, D)
    w_spec = pl.BlockSpec((D, D), lambda g: (0, 0))
    x_spec = pl.BlockSpec((1, L, D), lambda g: (g, 0, 0))
    return pl.pallas_call(
        _mixqkv_body,
        grid=(Bq,),
        in_specs=[x_spec, x_spec, w_spec, w_spec, w_spec, w_spec],
        out_specs=[x_spec, x_spec, x_spec],
        out_shape=[jax.ShapeDtypeStruct((Bq, L, D), jnp.float32)] * 3,
    )(h3, t3, Wm, Wq, Wk, Wv)


_NW = 32
_GRW = E // _NW
_GADD = None


def _gadd_body(vh_hbm, si_hbm, di_hbm, out_hbm, ia, ib, bufa, bufb, sema, semb):
    c = lax.axis_index("c")
    s = lax.axis_index("s")
    w = s * 2 + c
    pltpu.sync_copy(si_hbm.at[w], ia)
    pltpu.sync_copy(di_hbm.at[w], ib)
    cps = []
    for j in range(_GRW // 128):
        cps.append(pltpu.async_copy(vh_hbm.at[ia.at[j]], bufa.at[pl.ds(j * 128, 128)], sema))
    for cp in cps:
        cp.wait()
    cps = []
    for j in range(_GRW // 128):
        cps.append(pltpu.async_copy(vh_hbm.at[ib.at[j]], bufa.at[pl.ds(j * 128, 128)], semb, add=True))
    for cp in cps:
        cp.wait()
    pltpu.sync_copy(bufa, out_hbm.at[pl.ds(w * _GRW, _GRW)])


def _gather_add(vh, src3, dst3):
    global _GADD
    if _GADD is None:
        _GADD = pl.kernel(
            _gadd_body,
            out_type=jax.ShapeDtypeStruct((E, D), jnp.float32),
            mesh=plsc.VectorSubcoreMesh(core_axis_name="c", subcore_axis_name="s"),
            scratch_types=[
                pltpu.VMEM((_GRW // 128, 128), jnp.int32),
                pltpu.VMEM((_GRW // 128, 128), jnp.int32),
                pltpu.VMEM((_GRW, D), jnp.float32),
                pltpu.VMEM((_GRW, D), jnp.float32),
                pltpu.SemaphoreType.DMA,
                pltpu.SemaphoreType.DMA,
            ],
        )
    return _GADD(vh, src3, dst3)


def _att_core(q, k, v):
    Bq, L, _ = q.shape
    qh = q.reshape(Bq, L, H, DH).transpose(0, 2, 1, 3)
    kh = k.reshape(Bq, L, H, DH).transpose(0, 2, 1, 3)
    vh = v.reshape(Bq, L, H, DH).transpose(0, 2, 1, 3)
    scores = jnp.einsum('bhqd,bhkd->bhqk', qh, kh) / jnp.sqrt(jnp.float32(DH))
    att = jax.nn.softmax(scores, axis=-1)
    out = jnp.einsum('bhqk,bhkd->bhqd', att, vh).transpose(0, 2, 1, 3).reshape(Bq, L, D)
    return out


def _wo_body(x_ref, wo_ref, o_ref):
    o_ref[0] = x_ref[0] @ wo_ref[...]


def _wo_mm(xd, Wo):
    Bq, L, _ = xd.shape
    x_spec = pl.BlockSpec((1, L, D), lambda g: (g, 0, 0))
    return pl.pallas_call(
        _wo_body,
        grid=(Bq,),
        in_specs=[x_spec, pl.BlockSpec((D, D), lambda g: (0, 0))],
        out_specs=x_spec,
        out_shape=jax.ShapeDtypeStruct((Bq, L, D), jnp.float32),
    )(xd, Wo)


def _pool_body(K, x_ref, s_ref, sub_ref, perm_ref):
    L = x_ref.shape[1]
    x = x_ref[0]
    s_col = s_ref[0]
    s_row = jnp.transpose(s_col)
    sc = jnp.broadcast_to(s_col, (L, L))
    sr = jnp.broadcast_to(s_row, (L, L))
    ii = jax.lax.broadcasted_iota(jnp.int32, (L, L), 1)
    jj = jax.lax.broadcasted_iota(jnp.int32, (L, L), 0)
    beats = (sr > sc) | ((sr == sc) & (ii < jj))
    rank_col = jnp.sum(beats.astype(jnp.float32), axis=1, keepdims=True)
    r_lk = jax.lax.broadcasted_iota(jnp.int32, (L, K), 1).astype(jnp.float32)
    oh = (jnp.broadcast_to(rank_col, (L, K)) == r_lk).astype(jnp.float32)
    sub = _tn(oh, x)
    vals = _tn(oh, s_col)
    iota_col = jax.lax.broadcasted_iota(jnp.int32, (L, 1), 0).astype(jnp.float32)
    perm = _tn(oh, iota_col)
    sub_ref[0] = sub * jnp.tanh(vals)
    perm_ref[0] = perm.astype(jnp.int32)


def _pool(xd, scores, K):
    Bq, L, _ = xd.shape
    return pl.pallas_call(
        functools.partial(_pool_body, K),
        grid=(Bq,),
        in_specs=[pl.BlockSpec((1, L, D), lambda g: (g, 0, 0)),
                  pl.BlockSpec((1, L, 1), lambda g: (g, 0, 0))],
        out_specs=[pl.BlockSpec((1, K, D), lambda g: (g, 0, 0)),
                   pl.BlockSpec((1, K, 1), lambda g: (g, 0, 0))],
        out_shape=[jax.ShapeDtypeStruct((Bq, K, D), jnp.float32),
                   jax.ShapeDtypeStruct((Bq, K, 1), jnp.int32)],
    )(xd, scores)


def _encoder(xd, Wq, Wk, Wv, Wo):
    q, k, v = _qkv(xd, Wq, Wk, Wv)
    mid = _att_core(q, k, v)
    return _wo_mm(mid, Wo)


def _poolqkv_body(K, x_ref, s_ref, wq_ref, wk_ref, wv_ref,
                  sub_ref, perm_ref, q_ref, k_ref, v_ref):
    _pool_body(K, x_ref, s_ref, sub_ref, perm_ref)
    x = x_ref[0]
    q_ref[0] = x @ wq_ref[...]
    k_ref[0] = x @ wk_ref[...]
    v_ref[0] = x @ wv_ref[...]


def _pool_qkv(xd, scores, K, Wq, Wk, Wv):
    Bq, L, _ = xd.shape
    w_spec = pl.BlockSpec((D, D), lambda g: (0, 0))
    x_spec = pl.BlockSpec((1, L, D), lambda g: (g, 0, 0))
    return pl.pallas_call(
        functools.partial(_poolqkv_body, K),
        grid=(Bq,),
        in_specs=[x_spec,
                  pl.BlockSpec((1, L, 1), lambda g: (g, 0, 0)),
                  w_spec, w_spec, w_spec],
        out_specs=[pl.BlockSpec((1, K, D), lambda g: (g, 0, 0)),
                   pl.BlockSpec((1, K, 1), lambda g: (g, 0, 0)),
                   x_spec, x_spec, x_spec],
        out_shape=[jax.ShapeDtypeStruct((Bq, K, D), jnp.float32),
                   jax.ShapeDtypeStruct((Bq, K, 1), jnp.int32),
                   jax.ShapeDtypeStruct((Bq, L, D), jnp.float32),
                   jax.ShapeDtypeStruct((Bq, L, D), jnp.float32),
                   jax.ShapeDtypeStruct((Bq, L, D), jnp.float32)],
    )(xd, scores, Wq, Wk, Wv)


def _score(enc, pv):
    pn = pv / (jnp.linalg.norm(pv) + 1e-12)
    return (enc @ pn)[..., None]


def kernel(sparse_x, edge_index, batch, e_x, e_edge_index, e_batch,
           Wv_g, We_g, Wev, Wve, Wq, Wk, Wva, Wo, p):
    src, dst = edge_index[0], edge_index[1]
    esrc, edst = e_edge_index[0], e_edge_index[1]

    v_agg = jax.ops.segment_sum(sparse_x[src], dst, num_segments=N)
    e_agg = jax.ops.segment_sum(e_x[esrc], edst, num_segments=E)
    v_h = _gcn_mm(sparse_x, v_agg, dst, Wv_g, NP, EP)
    e_h = _gcn_mm(e_x, e_agg, edst, We_g, EP, 2 * EP)

    t_v = jax.ops.segment_sum(e_h, src, num_segments=N) + jax.ops.segment_sum(e_h, dst, num_segments=N)
    g2 = _gather_add(v_h, src.reshape(_NW, _GRW // 128, 128), dst.reshape(_NW, _GRW // 128, 128))

    qv, kv, vv = _mix_qkv(v_h, t_v, Wev, Wq[0], Wk[0], Wva[0], NP)
    enc1 = _wo_mm(_att_core(qv, kv, vv), Wo[0])
    v1, p1, q2v, k2v, v2v = _pool_qkv(enc1, _score(enc1, p[0]), KV, Wq[1], Wk[1], Wva[1])
    enc2 = _wo_mm(_att_core(q2v, k2v, v2v), Wo[1])
    v2, p2 = _pool(enc2, _score(enc2, p[1]), KV)
    qe, ke, ve = _mix_qkv(e_h, g2, Wve, Wq[2], Wk[2], Wva[2], EP)
    ee1 = _wo_mm(_att_core(qe, ke, ve), Wo[2])
    e1, q1, q4v, k4v, v4v = _pool_qkv(ee1, _score(ee1, p[2]), KE, Wq[3], Wk[3], Wva[3])
    ee2 = _wo_mm(_att_core(q4v, k4v, v4v), Wo[3])
    e2, q2 = _pool(ee2, _score(ee2, p[3]), KE)

    out = jnp.concatenate([v1.reshape(-1, D), v2.reshape(-1, D),
                           e1.reshape(-1, D), e2.reshape(-1, D)], axis=0)
    return (out, p1.reshape(B, KV), p2.reshape(B, KV),
            q1.reshape(B, KE), q2.reshape(B, KE))
